# baseline (speedup 1.0000x reference)
import functools

import jax
import jax.numpy as jnp
from jax import lax
from jax.experimental import pallas as pl
from jax.experimental.pallas import tpu as pltpu
from jax.experimental.pallas import tpu_sc as plsc

_VOCAB = 1000000
_EMB = 64
_CTX = 200

_NC = 2
_NS = 16
_ROWS_PER_W = 8
_ACTIVE_W = _CTX // _ROWS_PER_W

_BV = 8000
_NB = _VOCAB // _BV
_BF = 25000
_NF = _VOCAB // _BF


def _sc_gather(idx, table):
  mesh = plsc.VectorSubcoreMesh(core_axis_name="c", subcore_axis_name="s",
                                num_cores=_NC, num_subcores=_NS)

  @functools.partial(
      pl.kernel, mesh=mesh,
      out_type=jax.ShapeDtypeStruct((_CTX, _EMB), jnp.float32),
      scratch_types=[
          pltpu.VMEM((_ROWS_PER_W,), jnp.int32),
          pltpu.VMEM((_ROWS_PER_W, _EMB), jnp.float32),
          pltpu.SemaphoreType.DMA,
      ],
      compiler_params=pltpu.CompilerParams(use_tc_tiling_on_sc=False),
  )
  def gather_kernel(idx_hbm, table_hbm, out_hbm, idx_v, rows_v, sem):
    wid = lax.axis_index("s") * _NC + lax.axis_index("c")

    @pl.when(wid < _ACTIVE_W)
    def _():
      base = wid * _ROWS_PER_W
      pltpu.sync_copy(idx_hbm.at[pl.ds(base, _ROWS_PER_W)], idx_v)
      pltpu.async_copy(table_hbm.at[idx_v], rows_v, sem).wait()
      pltpu.sync_copy(rows_v, out_hbm.at[pl.ds(base, _ROWS_PER_W)])

  return gather_kernel(idx, table)


def _matvec_body(rows_ref, w_ref, b_ref, out_ref, lse_ref, m_ref, s_ref):
  i = pl.program_id(0)
  bow = jnp.sum(rows_ref[...], axis=0, keepdims=True)
  out = lax.dot_general(bow, w_ref[0], (((1,), (1,)), ((), ())),
                        preferred_element_type=jnp.float32)
  out = out + b_ref[0]
  out_ref[0] = out
  bm = jnp.max(out, keepdims=True)

  @pl.when(i == 0)
  def _():
    m_ref[...] = bm
    s_ref[...] = jnp.sum(jnp.exp(out - bm), keepdims=True)

  @pl.when(i > 0)
  def _():
    m_old = m_ref[...]
    m_new = jnp.maximum(m_old, bm)
    s_ref[...] = (s_ref[...] * jnp.exp(m_old - m_new)
                  + jnp.sum(jnp.exp(out - m_new), keepdims=True))
    m_ref[...] = m_new

  @pl.when(i == _NB - 1)
  def _():
    lse_ref[...] = m_ref[...] + jnp.log(s_ref[...])


def _finalize_body(out_raw_ref, lse_ref, lp_ref):
  lp_ref[0] = out_raw_ref[0] - lse_ref[...]


def _matvec(rows, w3, b3, *, interpret=False):
  return pl.pallas_call(
      _matvec_body,
      grid=(_NB,),
      in_specs=[
          pl.BlockSpec((_CTX, _EMB), lambda i: (0, 0)),
          pl.BlockSpec((1, _BV, _EMB), lambda i: (i, 0, 0)),
          pl.BlockSpec((1, 1, _BV), lambda i: (i, 0, 0)),
      ],
      out_specs=[
          pl.BlockSpec((1, 1, _BV), lambda i: (i, 0, 0)),
          pl.BlockSpec((1, 1), lambda i: (0, 0)),
      ],
      out_shape=[
          jax.ShapeDtypeStruct((_NB, 1, _BV), jnp.float32),
          jax.ShapeDtypeStruct((1, 1), jnp.float32),
      ],
      scratch_shapes=[
          pltpu.VMEM((1, 1), jnp.float32),
          pltpu.VMEM((1, 1), jnp.float32),
      ],
      interpret=interpret,
  )(rows, w3, b3)


def _finalize(out3, lse, *, interpret=False):
  return pl.pallas_call(
      _finalize_body,
      grid=(_NF,),
      in_specs=[
          pl.BlockSpec((1, 1, _BF), lambda i: (i, 0, 0)),
          pl.BlockSpec((1, 1), lambda i: (0, 0)),
      ],
      out_specs=pl.BlockSpec((1, 1, _BF), lambda i: (i, 0, 0)),
      out_shape=jax.ShapeDtypeStruct((_NF, 1, _BF), jnp.float32),
      interpret=interpret,
  )(out3, lse)


def kernel(input, emb_table, W, b):
  idx = input.astype(jnp.int32)
  rows = _sc_gather(idx, emb_table)
  w3 = W.reshape(_NB, _BV, _EMB)
  b3 = b.reshape(_NB, 1, _BV)
  out3, lse = _matvec(rows, w3, b3)
  lp3 = _finalize(out3.reshape(_NF, 1, _BF), lse)
  return lp3.reshape(1, _VOCAB)


# speedup vs baseline: 5.9334x; 5.9334x over previous
import jax
import jax.numpy as jnp
from jax import lax
from jax.experimental import pallas as pl
from jax.experimental.pallas import tpu as pltpu

_VOCAB = 1000000
_EMB = 64
_CTX = 200

_NSEM = 16
_BV = 8192
_NB = (_VOCAB + _BV - 1) // _BV
_BF = 65536
_NF = (_VOCAB + _BF - 1) // _BF


def _gather_body(idx_ref, et_ref, bow_ref, buf, sems):
  def _copy(j):
    base = pl.multiple_of((idx_ref[j] // 128) * 128, 128)
    return pltpu.make_async_copy(
        et_ref.at[:, pl.ds(base, 128)],
        buf.at[j],
        sems.at[j % _NSEM],
    )

  def _issue(j, carry):
    _copy(j).start()
    return carry

  def _drain(j, carry):
    _copy(j).wait()
    return carry

  lax.fori_loop(0, _CTX, _issue, 0)
  lax.fori_loop(0, _CTX, _drain, 0)

  lanes = lax.broadcasted_iota(jnp.int32, (_EMB, 128), 1)

  def _acc(j, acc128):
    lane = idx_ref[j] % 128
    return acc128 + jnp.where(lanes == lane, buf[j], 0.0)

  acc128 = lax.fori_loop(0, _CTX, _acc,
                         jnp.zeros((_EMB, 128), jnp.float32))
  bow_ref[...] = jnp.sum(acc128, axis=1, keepdims=True)


def _gather_pool(idx, et):
  return pl.pallas_call(
      _gather_body,
      in_specs=[
          pl.BlockSpec(memory_space=pltpu.SMEM),
          pl.BlockSpec(memory_space=pl.ANY),
      ],
      out_specs=pl.BlockSpec(memory_space=pltpu.VMEM),
      out_shape=jax.ShapeDtypeStruct((_EMB, 1), jnp.float32),
      scratch_shapes=[
          pltpu.VMEM((_CTX, _EMB, 128), jnp.float32),
          pltpu.SemaphoreType.DMA((_NSEM,)),
      ],
  )(idx, et)


def _matvec_body(bow_ref, wt_ref, b_ref, out_ref, lse_ref, m_ref, s_ref):
  i = pl.program_id(0)
  out = lax.dot_general(bow_ref[...], wt_ref[...], (((0,), (0,)), ((), ())),
                        preferred_element_type=jnp.float32)
  out = out + b_ref[...]
  out_ref[...] = out
  lane = lax.broadcasted_iota(jnp.int32, (1, _BV), 1)
  outm = jnp.where(lane < _VOCAB - i * _BV, out, -jnp.inf)
  bm = jnp.max(outm, keepdims=True)

  @pl.when(i == 0)
  def _():
    m_ref[...] = bm
    s_ref[...] = jnp.sum(jnp.exp(outm - bm), keepdims=True)

  @pl.when(i > 0)
  def _():
    m_old = m_ref[...]
    m_new = jnp.maximum(m_old, bm)
    s_ref[...] = (s_ref[...] * jnp.exp(m_old - m_new)
                  + jnp.sum(jnp.exp(outm - m_new), keepdims=True))
    m_ref[...] = m_new

  @pl.when(i == _NB - 1)
  def _():
    lse_ref[...] = m_ref[...] + jnp.log(s_ref[...])


def _matvec(bow, wt, b2):
  return pl.pallas_call(
      _matvec_body,
      grid=(_NB,),
      in_specs=[
          pl.BlockSpec((_EMB, 1), lambda i: (0, 0)),
          pl.BlockSpec((_EMB, _BV), lambda i: (0, i)),
          pl.BlockSpec((1, _BV), lambda i: (0, i)),
      ],
      out_specs=[
          pl.BlockSpec((1, _BV), lambda i: (0, i)),
          pl.BlockSpec((1, 1), lambda i: (0, 0)),
      ],
      out_shape=[
          jax.ShapeDtypeStruct((1, _VOCAB), jnp.float32),
          jax.ShapeDtypeStruct((1, 1), jnp.float32),
      ],
      scratch_shapes=[
          pltpu.VMEM((1, 1), jnp.float32),
          pltpu.VMEM((1, 1), jnp.float32),
      ],
  )(bow, wt, b2)


def _finalize_body(out_raw_ref, lse_ref, lp_ref):
  lp_ref[...] = out_raw_ref[...] - lse_ref[...]


def _finalize(out_raw, lse):
  return pl.pallas_call(
      _finalize_body,
      grid=(_NF,),
      in_specs=[
          pl.BlockSpec((1, _BF), lambda i: (0, i)),
          pl.BlockSpec((1, 1), lambda i: (0, 0)),
      ],
      out_specs=pl.BlockSpec((1, _BF), lambda i: (0, i)),
      out_shape=jax.ShapeDtypeStruct((1, _VOCAB), jnp.float32),
  )(out_raw, lse)


def kernel(input, emb_table, W, b):
  idx = input.astype(jnp.int32)
  et = emb_table.T
  wt = W.T
  bow = _gather_pool(idx, et)
  out_raw, lse = _matvec(bow, wt, b.reshape(1, _VOCAB))
  return _finalize(out_raw, lse)


# speedup vs baseline: 11.1904x; 1.8860x over previous
import jax
import jax.numpy as jnp
from jax import lax
from jax.experimental import pallas as pl
from jax.experimental.pallas import tpu as pltpu

_VOCAB = 1000000
_EMB = 64
_CTX = 200

_NSEM = 16
_BV = 32768
_NB = (_VOCAB + _BV - 1) // _BV
_BF = 65536
_NF = (_VOCAB + _BF - 1) // _BF


def _gather_body(idx_ref, et_ref, bow_ref, buf, sems):
  def _copy(j):
    base = pl.multiple_of((idx_ref[j] // 128) * 128, 128)
    return pltpu.make_async_copy(
        et_ref.at[:, pl.ds(base, 128)],
        buf.at[j],
        sems.at[j % _NSEM],
    )

  def _issue(j, carry):
    _copy(j).start()
    return carry

  def _drain(j, carry):
    _copy(j).wait()
    return carry

  lax.fori_loop(0, _CTX, _issue, 0)
  lax.fori_loop(0, _CTX, _drain, 0)

  lanes = lax.broadcasted_iota(jnp.int32, (_EMB, 128), 1)

  def _acc(j, acc128):
    lane = idx_ref[j] % 128
    return acc128 + jnp.where(lanes == lane, buf[j], 0.0)

  acc128 = lax.fori_loop(0, _CTX, _acc,
                         jnp.zeros((_EMB, 128), jnp.float32))
  bow_ref[...] = jnp.sum(acc128, axis=1, keepdims=True)


def _gather_pool(idx, et):
  return pl.pallas_call(
      _gather_body,
      in_specs=[
          pl.BlockSpec(memory_space=pltpu.SMEM),
          pl.BlockSpec(memory_space=pl.ANY),
      ],
      out_specs=pl.BlockSpec(memory_space=pltpu.VMEM),
      out_shape=jax.ShapeDtypeStruct((_EMB, 1), jnp.float32),
      scratch_shapes=[
          pltpu.VMEM((_CTX, _EMB, 128), jnp.float32),
          pltpu.SemaphoreType.DMA((_NSEM,)),
      ],
  )(idx, et)


def _matvec_body(bow_ref, wt_ref, b_ref, out_ref, lse_ref, m_ref, s_ref):
  i = pl.program_id(0)
  out = lax.dot_general(bow_ref[...], wt_ref[...], (((0,), (0,)), ((), ())),
                        preferred_element_type=jnp.float32)
  out = out + b_ref[...][None, :]
  out_ref[...] = out
  lane = lax.broadcasted_iota(jnp.int32, (1, _BV), 1)
  outm = jnp.where(lane < _VOCAB - i * _BV, out, -jnp.inf)
  bm = jnp.max(outm, keepdims=True)

  @pl.when(i == 0)
  def _():
    m_ref[...] = bm
    s_ref[...] = jnp.sum(jnp.exp(outm - bm), keepdims=True)

  @pl.when(i > 0)
  def _():
    m_old = m_ref[...]
    m_new = jnp.maximum(m_old, bm)
    s_ref[...] = (s_ref[...] * jnp.exp(m_old - m_new)
                  + jnp.sum(jnp.exp(outm - m_new), keepdims=True))
    m_ref[...] = m_new

  @pl.when(i == _NB - 1)
  def _():
    lse_ref[...] = m_ref[...] + jnp.log(s_ref[...])


def _matvec(bow, wt, b1):
  return pl.pallas_call(
      _matvec_body,
      grid=(_NB,),
      in_specs=[
          pl.BlockSpec((_EMB, 1), lambda i: (0, 0)),
          pl.BlockSpec((_EMB, _BV), lambda i: (0, i)),
          pl.BlockSpec((_BV,), lambda i: (i,)),
      ],
      out_specs=[
          pl.BlockSpec((1, _BV), lambda i: (0, i)),
          pl.BlockSpec((1, 1), lambda i: (0, 0)),
      ],
      out_shape=[
          jax.ShapeDtypeStruct((1, _VOCAB), jnp.float32),
          jax.ShapeDtypeStruct((1, 1), jnp.float32),
      ],
      scratch_shapes=[
          pltpu.VMEM((1, 1), jnp.float32),
          pltpu.VMEM((1, 1), jnp.float32),
      ],
  )(bow, wt, b1)


def _finalize_body(out_raw_ref, lse_ref, lp_ref):
  lp_ref[...] = out_raw_ref[...] - lse_ref[...]


def _finalize(out_raw, lse):
  return pl.pallas_call(
      _finalize_body,
      grid=(_NF,),
      in_specs=[
          pl.BlockSpec((1, _BF), lambda i: (0, i)),
          pl.BlockSpec((1, 1), lambda i: (0, 0)),
      ],
      out_specs=pl.BlockSpec((1, _BF), lambda i: (0, i)),
      out_shape=jax.ShapeDtypeStruct((1, _VOCAB), jnp.float32),
  )(out_raw, lse)


def kernel(input, emb_table, W, b):
  idx = input.astype(jnp.int32)
  et = emb_table.T
  wt = W.T
  bow = _gather_pool(idx, et)
  out_raw, lse = _matvec(bow, wt, b)
  return _finalize(out_raw, lse)


# speedup vs baseline: 11.5589x; 1.0329x over previous
import jax
import jax.numpy as jnp
from jax import lax
from jax.experimental import pallas as pl
from jax.experimental.pallas import tpu as pltpu

_VOCAB = 1000000
_EMB = 64
_CTX = 200

_NSEM = 16
_BV = 65536
_NB = (_VOCAB + _BV - 1) // _BV
_BF = 131072
_NF = (_VOCAB + _BF - 1) // _BF


def _gather_body(idx_ref, et_ref, bow_ref, buf, sems):
  def _copy(j):
    base = pl.multiple_of((idx_ref[j] // 128) * 128, 128)
    return pltpu.make_async_copy(
        et_ref.at[:, pl.ds(base, 128)],
        buf.at[j],
        sems.at[j % _NSEM],
    )

  def _issue(j, carry):
    _copy(j).start()
    return carry

  def _drain(j, carry):
    _copy(j).wait()
    return carry

  lax.fori_loop(0, _CTX, _issue, 0)
  lax.fori_loop(0, _CTX, _drain, 0)

  lanes = lax.broadcasted_iota(jnp.int32, (_EMB, 128), 1)

  def _acc(j, acc128):
    lane = idx_ref[j] % 128
    return acc128 + jnp.where(lanes == lane, buf[j], 0.0)

  acc128 = lax.fori_loop(0, _CTX, _acc,
                         jnp.zeros((_EMB, 128), jnp.float32))
  bow_ref[...] = jnp.sum(acc128, axis=1, keepdims=True)


def _gather_pool(idx, et):
  return pl.pallas_call(
      _gather_body,
      in_specs=[
          pl.BlockSpec(memory_space=pltpu.SMEM),
          pl.BlockSpec(memory_space=pl.ANY),
      ],
      out_specs=pl.BlockSpec(memory_space=pltpu.VMEM),
      out_shape=jax.ShapeDtypeStruct((_EMB, 1), jnp.float32),
      scratch_shapes=[
          pltpu.VMEM((_CTX, _EMB, 128), jnp.float32),
          pltpu.SemaphoreType.DMA((_NSEM,)),
      ],
  )(idx, et)


def _matvec_body(bow_ref, wt_ref, b_ref, out_ref, lse_ref, m_ref, s_ref):
  i = pl.program_id(0)
  out = lax.dot_general(bow_ref[...], wt_ref[...], (((0,), (0,)), ((), ())),
                        preferred_element_type=jnp.float32)
  out = out + b_ref[...][None, :]
  out_ref[...] = out
  lane = lax.broadcasted_iota(jnp.int32, (1, _BV), 1)
  outm = jnp.where(lane < _VOCAB - i * _BV, out, -jnp.inf)
  bm = jnp.max(outm, keepdims=True)

  @pl.when(i == 0)
  def _():
    m_ref[...] = bm
    s_ref[...] = jnp.sum(jnp.exp(outm - bm), keepdims=True)

  @pl.when(i > 0)
  def _():
    m_old = m_ref[...]
    m_new = jnp.maximum(m_old, bm)
    s_ref[...] = (s_ref[...] * jnp.exp(m_old - m_new)
                  + jnp.sum(jnp.exp(outm - m_new), keepdims=True))
    m_ref[...] = m_new

  @pl.when(i == _NB - 1)
  def _():
    lse_ref[...] = m_ref[...] + jnp.log(s_ref[...])


def _matvec(bow, wt, b1):
  return pl.pallas_call(
      _matvec_body,
      grid=(_NB,),
      in_specs=[
          pl.BlockSpec((_EMB, 1), lambda i: (0, 0)),
          pl.BlockSpec((_EMB, _BV), lambda i: (0, i)),
          pl.BlockSpec((_BV,), lambda i: (i,)),
      ],
      out_specs=[
          pl.BlockSpec((1, _BV), lambda i: (0, i)),
          pl.BlockSpec((1, 1), lambda i: (0, 0)),
      ],
      out_shape=[
          jax.ShapeDtypeStruct((1, _VOCAB), jnp.float32),
          jax.ShapeDtypeStruct((1, 1), jnp.float32),
      ],
      scratch_shapes=[
          pltpu.VMEM((1, 1), jnp.float32),
          pltpu.VMEM((1, 1), jnp.float32),
      ],
  )(bow, wt, b1)


def _finalize_body(out_raw_ref, lse_ref, lp_ref):
  lp_ref[...] = out_raw_ref[...] - lse_ref[...]


def _finalize(out_raw, lse):
  return pl.pallas_call(
      _finalize_body,
      grid=(_NF,),
      in_specs=[
          pl.BlockSpec((1, _BF), lambda i: (0, i)),
          pl.BlockSpec((1, 1), lambda i: (0, 0)),
      ],
      out_specs=pl.BlockSpec((1, _BF), lambda i: (0, i)),
      out_shape=jax.ShapeDtypeStruct((1, _VOCAB), jnp.float32),
  )(out_raw, lse)


def kernel(input, emb_table, W, b):
  idx = input.astype(jnp.int32)
  et = emb_table.T
  wt = W.T
  bow = _gather_pool(idx, et)
  out_raw, lse = _matvec(bow, wt, b)
  return _finalize(out_raw, lse)
